# baseline (speedup 1.0000x reference)
import jax
import jax.numpy as jnp
from jax import lax
from jax.experimental import pallas as pl
from jax.experimental.pallas import tpu as pltpu
from jax.experimental.pallas import tpu_sc as plsc

_BSZ = 32768
_NCLS = 512
_NGRAPH = 16
_L = 16
_NSUB = 16
_CHUNK = _BSZ // _NSUB
_UNROLL = 4

_LN2 = 0.6931471805599453


def _ln_vec(v):
    bits = plsc.bitcast(v, jnp.int32)
    e = ((bits >> 23) & 0xFF) - 127
    mant = plsc.bitcast((bits & 0x7FFFFF) | (127 << 23), jnp.float32)
    u = (mant - 1.0) / (mant + 1.0)
    u2 = u * u
    poly = 1.0 + u2 * (1.0 / 3.0 + u2 * (1.0 / 5.0 + u2 * (1.0 / 7.0 + u2 * (1.0 / 9.0))))
    return e.astype(jnp.float32) * _LN2 + 2.0 * u * poly


def _sc_body(x_hbm, idx_hbm, out_hbm,
             idx_v, p_v, row_v, loss_st, out_st, buf_v, sh, sem_i, sem_x):
    s = lax.axis_index("s")

    cp_i = pltpu.make_async_copy(idx_hbm.at[pl.ds(s * _CHUNK, _CHUNK)], idx_v, sem_i)
    cp_x = pltpu.make_async_copy(x_hbm.at[pl.ds(s * _NCLS, _NCLS)], row_v, sem_x)
    cp_i.start()
    cp_x.start()
    cp_i.wait()

    p_v[...] = jnp.zeros((_L,), jnp.float32)
    ones = jnp.ones((_L,), jnp.float32)

    def _hist(i, carry):
        for j in range(_UNROLL):
            plsc.store_scatter(p_v, [idx_v[pl.ds((i * _UNROLL + j) * _L, _L)]], ones)
        return carry

    lax.fori_loop(0, _CHUNK // (_L * _UNROLL), _hist, 0)
    pltpu.sync_copy(p_v, sh.at[pl.ds(s * _L, _L)])

    cp_x.wait()

    def _rmax(i, mv):
        return jnp.maximum(mv, row_v[pl.ds(i * _L, _L)])

    m = jnp.max(lax.fori_loop(1, _NCLS // _L, _rmax, row_v[pl.ds(0, _L)]))
    mb = jnp.broadcast_to(m, (_L,))

    def _rsum(i, acc):
        return acc + jnp.exp(row_v[pl.ds(i * _L, _L)] - mb)

    acc = lax.fori_loop(0, _NCLS // _L, _rsum, jnp.zeros((_L,), jnp.float32))
    ssum = jnp.sum(acc)
    x0 = row_v[pl.ds(0, _L)][0]
    loss_st[...] = mb + _ln_vec(jnp.broadcast_to(ssum, (_L,))) \
        - jnp.broadcast_to(x0, (_L,))
    pltpu.sync_copy(loss_st, sh.at[pl.ds(_NSUB * _L + s * _L, _L)])

    plsc.subcore_barrier()

    @pl.when(s == 0)
    def _combine():
        pltpu.sync_copy(sh, buf_v)

        def _pmax(r, pres):
            return jnp.maximum(pres, buf_v[pl.ds(r * _L, _L)])

        pres = lax.fori_loop(1, _NSUB, _pmax, buf_v[pl.ds(0, _L)])
        strided = _NSUB * _L + jnp.arange(_L, dtype=jnp.int32) * _L
        loss = plsc.load_gather(buf_v, [strided])
        pf = (pres > 0.0).astype(jnp.float32)
        nv = jnp.broadcast_to(jnp.sum(pf), (_L,))
        meanv = jnp.broadcast_to(jnp.sum(loss * pf), (_L,)) / nv
        d = loss - meanv
        out_st[pl.ds(0, _L)] = jnp.broadcast_to(jnp.sum(d * d * pf), (_L,)) / (nv - 1.0)
        out_st[pl.ds(_L, _L)] = (nv == 1.0).astype(jnp.float32)
        pltpu.sync_copy(out_st, out_hbm)


_sc_call = pl.kernel(
    _sc_body,
    out_type=jax.ShapeDtypeStruct((2 * _L,), jnp.float32),
    mesh=plsc.VectorSubcoreMesh(core_axis_name="c", subcore_axis_name="s",
                                num_cores=1),
    compiler_params=pltpu.CompilerParams(needs_layout_passes=False),
    scratch_types=[
        pltpu.VMEM((_CHUNK,), jnp.int32),
        pltpu.VMEM((_L,), jnp.float32),
        pltpu.VMEM((_NCLS,), jnp.float32),
        pltpu.VMEM((_L,), jnp.float32),
        pltpu.VMEM((2 * _L,), jnp.float32),
        pltpu.VMEM((2 * _NSUB * _L,), jnp.float32),
        pltpu.VMEM_SHARED((2 * _NSUB * _L,), jnp.float32),
        pltpu.SemaphoreType.DMA,
        pltpu.SemaphoreType.DMA,
    ],
)


def kernel(x, graph_idx, device):
    idx = graph_idx.astype(jnp.int32)
    xs = x[:_NGRAPH].reshape(-1)
    out = _sc_call(xs, idx)
    return out[0], out[_L] == 1.0


# speedup vs baseline: 1.0154x; 1.0154x over previous
import jax
import jax.numpy as jnp
from jax import lax
from jax.experimental import pallas as pl
from jax.experimental.pallas import tpu as pltpu
from jax.experimental.pallas import tpu_sc as plsc

_BSZ = 32768
_NCLS = 512
_NGRAPH = 16
_L = 16
_T = 16
_RPT = _NGRAPH // _T
_CHUNK = _BSZ // _T
_UNROLL = 4

_LN2 = 0.6931471805599453


def _ln_vec(v):
    bits = plsc.bitcast(v, jnp.int32)
    e = ((bits >> 23) & 0xFF) - 127
    mant = plsc.bitcast((bits & 0x7FFFFF) | (127 << 23), jnp.float32)
    u = (mant - 1.0) / (mant + 1.0)
    u2 = u * u
    poly = 1.0 + u2 * (1.0 / 3.0 + u2 * (1.0 / 5.0 + u2 * (1.0 / 7.0 + u2 * (1.0 / 9.0))))
    return e.astype(jnp.float32) * _LN2 + 2.0 * u * poly


def _sc_body(x_hbm, idx_hbm, out_hbm,
             idx_v, p_v, row_v, loss_st, out_st, buf_v, sh, sem_i, sem_x):
    s = lax.axis_index("s")

    cp_i = pltpu.make_async_copy(idx_hbm.at[pl.ds(s * _CHUNK, _CHUNK)], idx_v, sem_i)
    cp_x = pltpu.make_async_copy(
        x_hbm.at[pl.ds(s * _RPT * _NCLS, _RPT * _NCLS)], row_v, sem_x)
    cp_i.start()
    cp_x.start()
    cp_i.wait()

    p_v[...] = jnp.zeros((_L,), jnp.float32)
    ones = jnp.ones((_L,), jnp.float32)

    def _hist(i, carry):
        for j in range(_UNROLL):
            plsc.store_scatter(p_v, [idx_v[pl.ds((i * _UNROLL + j) * _L, _L)]], ones)
        return carry

    lax.fori_loop(0, _CHUNK // (_L * _UNROLL), _hist, 0)
    pltpu.sync_copy(p_v, sh.at[pl.ds(s * _L, _L)])

    cp_x.wait()
    for r in range(_RPT):
        base = r * _NCLS

        def _rmax(i, mv):
            return jnp.maximum(mv, row_v[pl.ds(base + i * _L, _L)])

        m = jnp.max(lax.fori_loop(1, _NCLS // _L, _rmax, row_v[pl.ds(base, _L)]))
        mb = jnp.broadcast_to(m, (_L,))

        def _rsum(i, acc):
            return acc + jnp.exp(row_v[pl.ds(base + i * _L, _L)] - mb)

        acc = lax.fori_loop(0, _NCLS // _L, _rsum, jnp.zeros((_L,), jnp.float32))
        ssum = jnp.sum(acc)
        x0 = row_v[pl.ds(base, _L)][0]
        loss_st[...] = mb + _ln_vec(jnp.broadcast_to(ssum, (_L,))) \
            - jnp.broadcast_to(x0, (_L,))
        pltpu.sync_copy(loss_st, sh.at[pl.ds(_T * _L + (s * _RPT + r) * _L, _L)])

    plsc.subcore_barrier()

    @pl.when(s == 0)
    def _combine():
        pltpu.sync_copy(sh, buf_v)

        def _pmax(t, pres):
            return jnp.maximum(pres, buf_v[pl.ds(t * _L, _L)])

        pres = lax.fori_loop(1, _T, _pmax, buf_v[pl.ds(0, _L)])
        strided = _T * _L + jnp.arange(_L, dtype=jnp.int32) * _L
        loss = plsc.load_gather(buf_v, [strided])
        pf = (pres > 0.0).astype(jnp.float32)
        nv = jnp.broadcast_to(jnp.sum(pf), (_L,))
        meanv = jnp.broadcast_to(jnp.sum(loss * pf), (_L,)) / nv
        d = loss - meanv
        out_st[pl.ds(0, _L)] = jnp.broadcast_to(jnp.sum(d * d * pf), (_L,)) / (nv - 1.0)
        out_st[pl.ds(_L, _L)] = (nv == 1.0).astype(jnp.float32)
        pltpu.sync_copy(out_st, out_hbm)


_sc_call = pl.kernel(
    _sc_body,
    out_type=jax.ShapeDtypeStruct((2 * _L,), jnp.float32),
    mesh=plsc.VectorSubcoreMesh(core_axis_name="c", subcore_axis_name="s",
                                num_cores=1),
    compiler_params=pltpu.CompilerParams(needs_layout_passes=False),
    scratch_types=[
        pltpu.VMEM((_CHUNK,), jnp.int32),
        pltpu.VMEM((_L,), jnp.float32),
        pltpu.VMEM((_RPT * _NCLS,), jnp.float32),
        pltpu.VMEM((_L,), jnp.float32),
        pltpu.VMEM((2 * _L,), jnp.float32),
        pltpu.VMEM(((_T + _NGRAPH) * _L,), jnp.float32),
        pltpu.VMEM_SHARED(((_T + _NGRAPH) * _L,), jnp.float32),
        pltpu.SemaphoreType.DMA,
        pltpu.SemaphoreType.DMA,
    ],
)


def kernel(x, graph_idx, device):
    idx = graph_idx.astype(jnp.int32)
    xs = x[:_NGRAPH].reshape(-1)
    out = _sc_call(xs, idx)
    return out[0], out[_L] == 1.0


# speedup vs baseline: 1.0232x; 1.0077x over previous
import jax
import jax.numpy as jnp
from jax import lax
from jax.experimental import pallas as pl
from jax.experimental.pallas import tpu as pltpu
from jax.experimental.pallas import tpu_sc as plsc

_BSZ = 32768
_NCLS = 512
_NGRAPH = 16
_L = 16
_NSUB = 16
_CHUNK = _BSZ // _NSUB

_LN2 = 0.6931471805599453


def _ln_vec(v):
    bits = plsc.bitcast(v, jnp.int32)
    e = ((bits >> 23) & 0xFF) - 127
    mant = plsc.bitcast((bits & 0x7FFFFF) | (127 << 23), jnp.float32)
    u = (mant - 1.0) / (mant + 1.0)
    u2 = u * u
    poly = 1.0 + u2 * (1.0 / 3.0 + u2 * (1.0 / 5.0 + u2 * (1.0 / 7.0 + u2 * (1.0 / 9.0))))
    return e.astype(jnp.float32) * _LN2 + 2.0 * u * poly


def _sc_body(x_hbm, idx_hbm, var_hbm, flag_hbm,
             idx_v, p_v, row_v, loss_st, h1d, l1d, sh_hist, sh_loss):
    s = lax.axis_index("s")

    pltpu.sync_copy(idx_hbm.at[pl.ds(s * _CHUNK, _CHUNK)], idx_v)
    p_v[...] = jnp.zeros((_L,), jnp.int32)
    ones = jnp.ones((_L,), jnp.int32)

    def _hist(i, carry):
        plsc.store_scatter(p_v, [idx_v[pl.ds(i * _L, _L)]], ones)
        return carry

    lax.fori_loop(0, _CHUNK // _L, _hist, 0)
    pltpu.sync_copy(p_v, sh_hist.at[pl.ds(s * _L, _L)])

    pltpu.sync_copy(x_hbm.at[pl.ds(s * _NCLS, _NCLS)], row_v)

    def _rmax(i, mv):
        return jnp.maximum(mv, row_v[pl.ds(i * _L, _L)])

    m = jnp.max(lax.fori_loop(1, _NCLS // _L, _rmax, row_v[pl.ds(0, _L)]))
    mb = jnp.broadcast_to(m, (_L,))

    def _rsum(i, acc):
        return acc + jnp.exp(row_v[pl.ds(i * _L, _L)] - mb)

    acc = lax.fori_loop(0, _NCLS // _L, _rsum, jnp.zeros((_L,), jnp.float32))
    ssum = jnp.sum(acc)
    x0 = row_v[pl.ds(0, _L)][0]
    loss_st[...] = mb + _ln_vec(jnp.broadcast_to(ssum, (_L,))) \
        - jnp.broadcast_to(x0, (_L,))
    pltpu.sync_copy(loss_st, sh_loss.at[pl.ds(s * _L, _L)])

    plsc.subcore_barrier()

    @pl.when(s == 0)
    def _combine():
        pltpu.sync_copy(sh_hist, h1d)
        pltpu.sync_copy(sh_loss, l1d)

        def _pmax(r, pres):
            return jnp.maximum(pres, h1d[pl.ds(r * _L, _L)])

        pres = lax.fori_loop(1, _NSUB, _pmax, h1d[pl.ds(0, _L)])
        strided = jnp.arange(_L, dtype=jnp.int32) * _L
        loss = plsc.load_gather(l1d, [strided])
        pf = (pres > 0).astype(jnp.float32)
        nv = jnp.broadcast_to(jnp.sum(pf), (_L,))
        meanv = jnp.broadcast_to(jnp.sum(loss * pf), (_L,)) / nv
        d = loss - meanv
        loss_st[...] = jnp.broadcast_to(jnp.sum(d * d * pf), (_L,)) / (nv - 1.0)
        pltpu.sync_copy(loss_st, var_hbm)
        p_v[...] = (nv == 1.0).astype(jnp.int32)
        pltpu.sync_copy(p_v, flag_hbm)


_sc_call = pl.kernel(
    _sc_body,
    out_type=[
        jax.ShapeDtypeStruct((_L,), jnp.float32),
        jax.ShapeDtypeStruct((_L,), jnp.int32),
    ],
    mesh=plsc.VectorSubcoreMesh(core_axis_name="c", subcore_axis_name="s",
                                num_cores=1),
    compiler_params=pltpu.CompilerParams(needs_layout_passes=False),
    scratch_types=[
        pltpu.VMEM((_CHUNK,), jnp.int32),
        pltpu.VMEM((_L,), jnp.int32),
        pltpu.VMEM((_NCLS,), jnp.float32),
        pltpu.VMEM((_L,), jnp.float32),
        pltpu.VMEM((_NSUB * _L,), jnp.int32),
        pltpu.VMEM((_NSUB * _L,), jnp.float32),
        pltpu.VMEM_SHARED((_NSUB * _L,), jnp.int32),
        pltpu.VMEM_SHARED((_NSUB * _L,), jnp.float32),
    ],
)


def kernel(x, graph_idx, device):
    idx = graph_idx.astype(jnp.int32)
    xs = x[:_NGRAPH].reshape(-1)
    var, flag = _sc_call(xs, idx)
    return var[0], flag[0] == 1
